# Initial kernel scaffold; baseline (speedup 1.0000x reference)
import jax
import jax.numpy as jnp
from jax.experimental import pallas as pl


def kernel(x, tok_table, pos_table):
    raise NotImplementedError("write your pallas kernel here")


# speedup vs baseline: 5.2360x; 5.2360x over previous
import jax
import jax.numpy as jnp
from jax import lax
from jax.experimental import pallas as pl
from jax.experimental.pallas import tpu as pltpu
from jax.experimental.pallas import tpu_sc as plsc
import functools

VOCAB = 100000
EMB = 64
N_TOKENS = 200
BATCH = 4096

NC = 2
NS = 16
NW = NC * NS

TOK_TOTAL = BATCH * N_TOKENS
PER_W = TOK_TOTAL // NW
CHUNK = 128
NCHUNK = PER_W // CHUNK


def _emb_kernel(x_hbm, tok_hbm, pos_hbm, out_hbm,
                idx_v, pos_v, row_a, row_b, gsem):
    wid = lax.axis_index("s") * NC + lax.axis_index("c")

    pltpu.sync_copy(x_hbm.at[wid], idx_v)
    pltpu.sync_copy(pos_hbm, pos_v)

    def gather(j, buf):
        pltpu.async_copy(tok_hbm.at[idx_v.at[j]], buf, gsem)

    def wait_gather(buf):
        pltpu.make_async_copy(tok_hbm.at[idx_v.at[0]], buf, gsem).wait()

    def add_store(j, buf):
        phase = lax.rem(j * CHUNK, N_TOKENS)

        def body(r, _):
            pr = phase + r
            for c in range(0, EMB, 16):
                buf[r, pl.ds(c, 16)] = (
                    buf[r, pl.ds(c, 16)] + pos_v[pr, pl.ds(c, 16)]
                )
            return 0

        lax.fori_loop(0, CHUNK, body, 0, unroll=4)
        pltpu.sync_copy(buf, out_hbm.at[wid, j])

    gather(0, row_a)

    def step(jj, _):
        for parity, (buf, nbuf) in enumerate(((row_a, row_b),
                                              (row_b, row_a))):
            j = jj * 2 + parity

            @pl.when(j + 1 < NCHUNK)
            def _():
                gather(j + 1, nbuf)

            wait_gather(buf)
            add_store(j, buf)
        return 0

    lax.fori_loop(0, NCHUNK // 2, step, 0)


@jax.jit
def kernel(x, tok_table, pos_table):
    xw = x.reshape(NW, NCHUNK, CHUNK).astype(jnp.int32)
    pos2 = jnp.concatenate([pos_table, pos_table], axis=0)

    mesh = plsc.VectorSubcoreMesh(core_axis_name="c", subcore_axis_name="s",
                                  num_cores=NC, num_subcores=NS)
    out = pl.kernel(
        _emb_kernel,
        out_type=jax.ShapeDtypeStruct((NW, NCHUNK, CHUNK, EMB), jnp.float32),
        mesh=mesh,
        scratch_types=[
            pltpu.VMEM((NCHUNK, CHUNK), jnp.int32),
            pltpu.VMEM((2 * N_TOKENS, EMB), jnp.float32),
            pltpu.VMEM((CHUNK, EMB), jnp.float32),
            pltpu.VMEM((CHUNK, EMB), jnp.float32),
            pltpu.SemaphoreType.DMA,
        ],
        compiler_params=pltpu.CompilerParams(use_tc_tiling_on_sc=False),
    )(xw, tok_table, pos2)
    return out.reshape(BATCH, N_TOKENS, EMB)


# speedup vs baseline: 9.8952x; 1.8898x over previous
import jax
import jax.numpy as jnp
from jax import lax
from jax.experimental import pallas as pl
from jax.experimental.pallas import tpu as pltpu
from jax.experimental.pallas import tpu_sc as plsc

VOCAB = 100000
EMB = 64
N_TOKENS = 200
BATCH = 4096

NC = 2
NS = 16
NW = NC * NS
BBLK = BATCH // NW
PITCH = BBLK + 1


def _emb_kernel(xt_hbm, tok_hbm, pos_hbm, out_hbm,
                idx_v, pos_v, row_a, row_b, tr_a, tr_b,
                gsem_a, gsem_b, ssem_a, ssem_b):
    wid = lax.axis_index("s") * NC + lax.axis_index("c")

    pltpu.sync_copy(xt_hbm.at[:, wid], idx_v)
    pltpu.sync_copy(pos_hbm, pos_v)

    iota16 = lax.iota(jnp.int32, 16)
    e_base = [iota16 + (16 * c) for c in range(4)]

    def gather(t, buf, sem):
        pltpu.async_copy(tok_hbm.at[idx_v.at[t]], buf, sem)

    def wait_gather(buf, sem):
        pltpu.make_async_copy(tok_hbm.at[idx_v.at[0]], buf, sem).wait()

    def transpose_add(t, buf, tb):
        posv = [pos_v[t, pl.ds(16 * c, 16)] for c in range(4)]

        def body(r, _):
            rv = jnp.broadcast_to(r, (16,)).astype(jnp.int32)
            for c in range(4):
                v = buf[r, pl.ds(16 * c, 16)] + posv[c]
                plsc.store_scatter(tb, [e_base[c], rv], v)
            return 0

        lax.fori_loop(0, BBLK, body, 0, unroll=4)

    def store(t, tb, sem):
        for eo in range(8):
            pltpu.async_copy(tb.at[pl.ds(eo * 8, 8), pl.ds(0, BBLK)],
                             out_hbm.at[t, eo, wid], sem)

    def wait_store(tb, sem):
        for eo in range(8):
            pltpu.make_async_copy(tb.at[pl.ds(eo * 8, 8), pl.ds(0, BBLK)],
                                  out_hbm.at[0, 0, 0], sem).wait()

    bufs = ((row_a, tr_a, gsem_a, ssem_a), (row_b, tr_b, gsem_b, ssem_b))

    gather(0, row_a, gsem_a)

    def step(tt, _):
        for parity, (buf, tb, gsem, ssem) in enumerate(bufs):
            t = tt * 2 + parity
            obuf, _otb, ogsem, _ossem = bufs[1 - parity]

            @pl.when(t + 1 < N_TOKENS)
            def _():
                gather(t + 1, obuf, ogsem)

            wait_gather(buf, gsem)

            @pl.when(t >= 2)
            def _():
                wait_store(tb, ssem)

            transpose_add(t, buf, tb)
            store(t, tb, ssem)
        return 0

    lax.fori_loop(0, N_TOKENS // 2, step, 0)
    wait_store(tr_a, ssem_a)
    wait_store(tr_b, ssem_b)


@jax.jit
def kernel(x, tok_table, pos_table):
    xt = x.T.reshape(N_TOKENS, NW, BBLK).astype(jnp.int32)

    mesh = plsc.VectorSubcoreMesh(core_axis_name="c", subcore_axis_name="s",
                                  num_cores=NC, num_subcores=NS)
    out = pl.kernel(
        _emb_kernel,
        out_type=jax.ShapeDtypeStruct((N_TOKENS, EMB // 8, NW, 8, BBLK),
                                      jnp.float32),
        mesh=mesh,
        scratch_types=[
            pltpu.VMEM((N_TOKENS, BBLK), jnp.int32),
            pltpu.VMEM((N_TOKENS, EMB), jnp.float32),
            pltpu.VMEM((BBLK, EMB), jnp.float32),
            pltpu.VMEM((BBLK, EMB), jnp.float32),
            pltpu.VMEM((EMB, PITCH), jnp.float32),
            pltpu.VMEM((EMB, PITCH), jnp.float32),
            pltpu.SemaphoreType.DMA,
            pltpu.SemaphoreType.DMA,
            pltpu.SemaphoreType.DMA,
            pltpu.SemaphoreType.DMA,
        ],
        compiler_params=pltpu.CompilerParams(use_tc_tiling_on_sc=False,
                                             needs_layout_passes=False),
    )(xt, tok_table, pos_table)
    return out.transpose(2, 4, 0, 1, 3).reshape(BATCH, N_TOKENS, EMB)
